# speedup vs baseline: 3.8732x; 3.8732x over previous
import functools

import jax
import jax.numpy as jnp
from jax import lax
from jax.experimental import pallas as pl
from jax.experimental.pallas import tpu as pltpu
from jax.experimental.pallas import tpu_sc as plsc

N_ROWS = 320000
D = 128
NSEG = 1024
NC = 2
NS = 16
NW = NC * NS
ROWS_PER_W = N_ROWS // NW
CHUNK = 80
NCHUNK = ROWS_PER_W // CHUNK
ROWS_PER_TILE_OUT = NSEG // NS


def _sc_body(x_hbm, b_hbm, z_hbm, out_hbm, xbuf, ibuf, acc):
    c = lax.axis_index("c")
    s = lax.axis_index("s")
    wid = c * NS + s
    base_w = wid * ROWS_PER_W

    pltpu.sync_copy(z_hbm, acc.at[pl.ds(s * ROWS_PER_TILE_OUT, ROWS_PER_TILE_OUT)])
    plsc.subcore_barrier()

    def step(i, carry):
        base = base_w + i * CHUNK
        pltpu.sync_copy(x_hbm.at[pl.ds(base, CHUNK)], xbuf)
        pltpu.sync_copy(b_hbm.at[pl.ds(base, CHUNK)], ibuf)
        pltpu.sync_copy(xbuf, acc.at[ibuf], add=True)
        return carry

    lax.fori_loop(0, NCHUNK, step, 0)

    plsc.subcore_barrier()
    row0 = s * ROWS_PER_TILE_OUT
    pltpu.sync_copy(
        acc.at[pl.ds(row0, ROWS_PER_TILE_OUT)],
        out_hbm.at[pl.ds(c * NSEG + row0, ROWS_PER_TILE_OUT)],
    )


def _combine_body(p_ref, o_ref):
    o_ref[...] = p_ref[0] + p_ref[1]


def kernel(x, batch):
    batch = batch.astype(jnp.int32)
    zeros = jnp.zeros((ROWS_PER_TILE_OUT, D), jnp.float32)

    mesh = plsc.VectorSubcoreMesh(core_axis_name="c", subcore_axis_name="s")
    partials = pl.kernel(
        _sc_body,
        out_type=jax.ShapeDtypeStruct((NC * NSEG, D), jnp.float32),
        mesh=mesh,
        scratch_types=[
            pltpu.VMEM((CHUNK, D), jnp.float32),
            pltpu.VMEM((CHUNK,), jnp.int32),
            pltpu.VMEM_SHARED((NSEG, D), jnp.float32),
        ],
    )(x, batch, zeros)

    out = pl.pallas_call(
        _combine_body,
        out_shape=jax.ShapeDtypeStruct((NSEG, D), jnp.float32),
    )(partials.reshape(NC, NSEG, D))
    return out


# speedup vs baseline: 7.2158x; 1.8630x over previous
import functools

import jax
import jax.numpy as jnp
from jax import lax
from jax.experimental import pallas as pl
from jax.experimental.pallas import tpu as pltpu
from jax.experimental.pallas import tpu_sc as plsc

N_ROWS = 320000
D = 128
NSEG = 1024
NC = 2
NS = 16
NW = NC * NS
ROWS_PER_W = N_ROWS // NW
CHUNK = 80
NCHUNK = ROWS_PER_W // CHUNK
ROWS_PER_TILE_OUT = NSEG // NS


def _sc_body(x_hbm, b_hbm, z_hbm, out_hbm, xb0, xb1, ib0, ib1, acc, sem0, sem1):
    c = lax.axis_index("c")
    s = lax.axis_index("s")
    wid = c * NS + s
    base_w = wid * ROWS_PER_W

    xbufs = (xb0, xb1)
    ibufs = (ib0, ib1)
    sems = (sem0, sem1)

    def issue(i, b):
        base = base_w + i * CHUNK
        pltpu.make_async_copy(x_hbm.at[pl.ds(base, CHUNK)], xbufs[b], sems[b]).start()
        pltpu.make_async_copy(b_hbm.at[pl.ds(base, CHUNK)], ibufs[b], sems[b]).start()

    def wait_and_scatter(b):
        pltpu.make_async_copy(x_hbm.at[pl.ds(base_w, CHUNK)], xbufs[b], sems[b]).wait()
        pltpu.make_async_copy(b_hbm.at[pl.ds(base_w, CHUNK)], ibufs[b], sems[b]).wait()
        pltpu.sync_copy(xbufs[b], acc.at[ibufs[b]], add=True)

    issue(0, 0)
    issue(1, 1)
    pltpu.sync_copy(z_hbm, acc.at[pl.ds(s * ROWS_PER_TILE_OUT, ROWS_PER_TILE_OUT)])
    plsc.subcore_barrier()

    def outer(g, carry):
        for b in range(2):
            i = 2 * g + b
            wait_and_scatter(b)

            @pl.when(i + 2 < NCHUNK)
            def _():
                issue(i + 2, b)

        return carry

    lax.fori_loop(0, NCHUNK // 2, outer, 0)
    if NCHUNK % 2:
        wait_and_scatter(0)

    plsc.subcore_barrier()
    row0 = s * ROWS_PER_TILE_OUT
    pltpu.sync_copy(
        acc.at[pl.ds(row0, ROWS_PER_TILE_OUT)],
        out_hbm.at[pl.ds(c * NSEG + row0, ROWS_PER_TILE_OUT)],
    )


def _combine_body(p_ref, o_ref):
    o_ref[...] = p_ref[0] + p_ref[1]


def kernel(x, batch):
    batch = batch.astype(jnp.int32)
    zeros = jnp.zeros((ROWS_PER_TILE_OUT, D), jnp.float32)

    mesh = plsc.VectorSubcoreMesh(core_axis_name="c", subcore_axis_name="s")
    partials = pl.kernel(
        _sc_body,
        out_type=jax.ShapeDtypeStruct((NC * NSEG, D), jnp.float32),
        mesh=mesh,
        scratch_types=[
            pltpu.VMEM((CHUNK, D), jnp.float32),
            pltpu.VMEM((CHUNK, D), jnp.float32),
            pltpu.VMEM((CHUNK,), jnp.int32),
            pltpu.VMEM((CHUNK,), jnp.int32),
            pltpu.VMEM_SHARED((NSEG, D), jnp.float32),
            pltpu.SemaphoreType.DMA,
            pltpu.SemaphoreType.DMA,
        ],
    )(x, batch, zeros)

    out = pl.pallas_call(
        _combine_body,
        out_shape=jax.ShapeDtypeStruct((NSEG, D), jnp.float32),
    )(partials.reshape(NC, NSEG, D))
    return out
